# baseline (speedup 1.0000x reference)
import functools

import jax
import jax.numpy as jnp
import numpy as np
from jax import lax
from jax.experimental import pallas as pl
from jax.experimental.pallas import tpu as pltpu
from jax.experimental.pallas import tpu_sc as plsc

ROWS = 128
COLS = 100000

SC_LO = 79872
SC_HI = 98304
SC_COLS = SC_HI - SC_LO
SC_HALF = SC_COLS // 2

BLOCK_COLS = 2048
TC_BLOCKS = SC_LO // BLOCK_COLS + 1
TAIL_BLOCK = SC_HI // BLOCK_COLS

_TINY = np.float32(np.finfo(np.float32).tiny)
_LN2 = np.float32(0.6931472)
_KS1 = np.uint32(42)
_KS2 = np.uint32(42 ^ 0x1BD11BDA)


def _rotl(x, d):
    return (x << np.uint32(d)) | (x >> np.uint32(32 - d))


def _round4(x0, x1, rots):
    for r in rots:
        x0 = x0 + x1
        x1 = _rotl(x1, r)
        x1 = x1 ^ x0
    return x0, x1


def _threefry_bits(m):
    x0 = m
    x1 = _rotl(m, 13) ^ m
    x0, x1 = _round4(x0, x1, (15, 26, 6))
    x0, x1 = x0 + _KS1, x1 + (_KS2 + np.uint32(1))
    x0, x1 = _round4(x0, x1, (17, 29, 16, 24))
    x0, x1 = x0 + _KS2, x1 + np.uint32(2)
    x0, x1 = _round4(x0, x1, (13, 15, 26, 6))
    x1 = x1 + (_KS1 + np.uint32(3))
    x0, x1 = _round4(x0, x1, (17, 29, 16, 24))
    x0, x1 = x0 + _KS1, x1 + (_KS2 + np.uint32(4))
    x0, x1 = _round4(x0, x1, (13, 15, 26, 6))
    x0, x1 = x0 + _KS2, x1 + np.uint32(5)
    return x0 ^ x1


def _uniform_from_bits(bits):
    fbits = (bits >> np.uint32(9)) | np.uint32(0x3F800000)
    floats = lax.bitcast_convert_type(fbits, jnp.float32) - np.float32(1.0)
    return floats + _TINY


def _tc_kernel(logits_ref, val_ref, idx_ref, best_val, best_idx):
    j = pl.program_id(0)
    nblocks = pl.num_programs(0)
    jblk = jnp.where(j == nblocks - 1, TAIL_BLOCK, j)

    lane = lax.broadcasted_iota(jnp.int32, (ROWS, BLOCK_COLS), 1)
    row = lax.broadcasted_iota(jnp.int32, (ROWS, BLOCK_COLS), 0)
    col = jblk * BLOCK_COLS + lane
    m = (row * COLS + col + 42).astype(jnp.uint32)

    u = _uniform_from_bits(_threefry_bits(m))
    g = -jnp.log(-jnp.log(u))

    x = logits_ref[...] + g
    x = jnp.where(col < COLS, x, -jnp.inf)

    bm = jnp.max(x, axis=1, keepdims=True)
    bi = jnp.min(jnp.where(x == bm, col, jnp.int32(2**30)), axis=1,
                 keepdims=True)

    @pl.when(j == 0)
    def _():
        best_val[...] = bm
        best_idx[...] = bi

    @pl.when(j > 0)
    def _():
        upd = bm > best_val[...]
        best_val[...] = jnp.where(upd, bm, best_val[...])
        best_idx[...] = jnp.where(upd, bi, best_idx[...])

    @pl.when(j == nblocks - 1)
    def _():
        val_ref[...] = best_val[...]
        idx_ref[...] = best_idx[...]


def _neg_ln_u(u):
    ub = lax.bitcast_convert_type(u, jnp.int32)
    be = (ub >> np.int32(23)).astype(jnp.float32)
    y = (np.float32(126.5) - be) * _LN2
    for _ in range(4):
        y = y + np.float32(1.0) - u * jnp.exp(y)
    d = np.float32(1.0) - u
    p = jnp.full_like(u, np.float32(1.0 / 13.0))
    for kinv in range(12, 0, -1):
        p = p * d + np.float32(1.0 / kinv)
    return jnp.where(u >= np.float32(0.75), d * p, y)


def _ln_t(t):
    tb = lax.bitcast_convert_type(t, jnp.int32)
    be = (tb >> np.int32(23)).astype(jnp.float32)
    z = (be - np.float32(126.5)) * _LN2
    for _ in range(4):
        z = z - np.float32(1.0) + t * jnp.exp(-z)
    return z


_SC_MESH = plsc.VectorSubcoreMesh(core_axis_name="c", subcore_axis_name="s")


@functools.partial(
    pl.kernel,
    out_type=(
        jax.ShapeDtypeStruct((ROWS, 16), jnp.float32),
        jax.ShapeDtypeStruct((ROWS, 16), jnp.float32),
        jax.ShapeDtypeStruct((ROWS, 16), jnp.int32),
        jax.ShapeDtypeStruct((ROWS, 16), jnp.float32),
        jax.ShapeDtypeStruct((ROWS, 16), jnp.float32),
        jax.ShapeDtypeStruct((ROWS, 16), jnp.int32),
    ),
    mesh=_SC_MESH,
    compiler_params=pltpu.CompilerParams(use_tc_tiling_on_sc=True),
    scratch_types=[
        pltpu.VMEM((8, SC_HALF), jnp.float32),
        pltpu.VMEM((8, 16), jnp.float32),
        pltpu.VMEM((8, 16), jnp.float32),
        pltpu.VMEM((8, 16), jnp.int32),
    ],
)
def _sc_scan(logits_hbm, u0, l0, c0, u1, l1, c1, slab, bu, bl, bc):
    wid = lax.axis_index("s") * 2 + lax.axis_index("c")
    grp = wid // 2
    half = wid % 2
    row0 = grp * 8
    cbase = SC_LO + half * SC_HALF
    pltpu.sync_copy(
        logits_hbm.at[pl.ds(row0, 8), pl.ds(half * SC_HALF, SC_HALF)], slab)

    lanes = lax.iota(jnp.int32, 16)
    lanes_u = lanes.astype(jnp.uint32)

    for rr in range(8):
        nbase = (row0 + rr) * COLS + cbase + 42

        def step(k, carry, rr=rr, nbase=nbase, cbase=cbase):
            xb, colb, ub, lb = carry
            base = k * 32
            logit1 = slab[rr, pl.ds(base, 16)]
            logit2 = slab[rr, pl.ds(base + 16, 16)]
            m1 = (nbase + base).astype(jnp.uint32) + lanes_u
            m2 = m1 + np.uint32(16)
            v1 = _uniform_from_bits(_threefry_bits(m1))
            v2 = _uniform_from_bits(_threefry_bits(m2))
            xc1 = logit1 - _ln_t(_neg_ln_u(v1))
            xc2 = logit2 - _ln_t(_neg_ln_u(v2))
            colv1 = (cbase + base) + lanes

            win1 = xc1 > xb
            xb = jnp.where(win1, xc1, xb)
            colb = jnp.where(win1, colv1, colb)
            ub = jnp.where(win1, v1, ub)
            lb = jnp.where(win1, logit1, lb)
            win2 = xc2 > xb
            return (jnp.where(win2, xc2, xb),
                    jnp.where(win2, colv1 + np.int32(16), colb),
                    jnp.where(win2, v2, ub),
                    jnp.where(win2, logit2, lb))

        init = (jnp.full((16,), -3.4e38, jnp.float32),
                jnp.zeros((16,), jnp.int32),
                jnp.full((16,), _TINY, jnp.float32),
                jnp.zeros((16,), jnp.float32))
        xb, colb, ub, lb = lax.fori_loop(0, SC_HALF // 32, step, init)
        bu[rr, :] = ub
        bl[rr, :] = lb
        bc[rr, :] = colb

    @pl.when(half == 0)
    def _():
        pltpu.sync_copy(bu, u0.at[pl.ds(row0, 8), :])
        pltpu.sync_copy(bl, l0.at[pl.ds(row0, 8), :])
        pltpu.sync_copy(bc, c0.at[pl.ds(row0, 8), :])

    @pl.when(half == 1)
    def _():
        pltpu.sync_copy(bu, u1.at[pl.ds(row0, 8), :])
        pltpu.sync_copy(bl, l1.at[pl.ds(row0, 8), :])
        pltpu.sync_copy(bc, c1.at[pl.ds(row0, 8), :])


def _merge_kernel(tcv_ref, tci_ref, u0_ref, l0_ref, c0_ref,
                  u1_ref, l1_ref, c1_ref, out_ref):
    big = jnp.int32(2**30)
    x0 = l0_ref[...] - jnp.log(-jnp.log(u0_ref[...]))
    x1 = l1_ref[...] - jnp.log(-jnp.log(u1_ref[...]))
    bm0 = jnp.max(x0, axis=1, keepdims=True)
    bm1 = jnp.max(x1, axis=1, keepdims=True)
    bm = jnp.maximum(bm0, bm1)
    bi0 = jnp.min(jnp.where(x0 == bm, c0_ref[...], big), axis=1, keepdims=True)
    bi1 = jnp.min(jnp.where(x1 == bm, c1_ref[...], big), axis=1, keepdims=True)
    bi = jnp.minimum(bi0, bi1)

    tv = tcv_ref[...]
    ti = tci_ref[...]
    take_sc = (bm > tv) | ((bm == tv) & (bi < ti))
    out_ref[...] = jnp.where(take_sc, bi, ti)


@jax.jit
def kernel(logits):
    sc_in = lax.slice(logits, (0, SC_LO), (ROWS, SC_HI))
    sc_res = _sc_scan(sc_in)
    tcv, tci = pl.pallas_call(
        _tc_kernel,
        grid=(TC_BLOCKS,),
        in_specs=[pl.BlockSpec(
            (ROWS, BLOCK_COLS),
            lambda j: (0, jnp.where(j == TC_BLOCKS - 1, TAIL_BLOCK, j)))],
        out_specs=(pl.BlockSpec((ROWS, 1), lambda j: (0, 0)),
                   pl.BlockSpec((ROWS, 1), lambda j: (0, 0))),
        out_shape=(jax.ShapeDtypeStruct((ROWS, 1), jnp.float32),
                   jax.ShapeDtypeStruct((ROWS, 1), jnp.int32)),
        scratch_shapes=[
            pltpu.VMEM((ROWS, 1), jnp.float32),
            pltpu.VMEM((ROWS, 1), jnp.int32),
        ],
    )(logits)
    out = pl.pallas_call(
        _merge_kernel,
        out_shape=jax.ShapeDtypeStruct((ROWS, 1), jnp.int32),
    )(tcv, tci, *sc_res)
    return out.reshape(ROWS)


# speedup vs baseline: 1.0010x; 1.0010x over previous
import functools

import jax
import jax.numpy as jnp
import numpy as np
from jax import lax
from jax.experimental import pallas as pl
from jax.experimental.pallas import tpu as pltpu
from jax.experimental.pallas import tpu_sc as plsc

ROWS = 128
COLS = 100000

SC_LO = 79872
SC_HI = 98304
SC_COLS = SC_HI - SC_LO
SC_HALF = SC_COLS // 2

BLOCK_COLS = 2048
TC_BLOCKS = SC_LO // BLOCK_COLS + 1
TAIL_BLOCK = SC_HI // BLOCK_COLS

_TINY = np.float32(np.finfo(np.float32).tiny)
_LN2 = np.float32(0.6931472)
_KS1 = np.uint32(42)
_KS2 = np.uint32(42 ^ 0x1BD11BDA)


def _rotl(x, d):
    return (x << np.uint32(d)) | (x >> np.uint32(32 - d))


def _round4(x0, x1, rots):
    for r in rots:
        x0 = x0 + x1
        x1 = _rotl(x1, r)
        x1 = x1 ^ x0
    return x0, x1


def _threefry_bits(m):
    x0 = m
    x1 = _rotl(m, 13) ^ m
    x0, x1 = _round4(x0, x1, (15, 26, 6))
    x0, x1 = x0 + _KS1, x1 + (_KS2 + np.uint32(1))
    x0, x1 = _round4(x0, x1, (17, 29, 16, 24))
    x0, x1 = x0 + _KS2, x1 + np.uint32(2)
    x0, x1 = _round4(x0, x1, (13, 15, 26, 6))
    x1 = x1 + (_KS1 + np.uint32(3))
    x0, x1 = _round4(x0, x1, (17, 29, 16, 24))
    x0, x1 = x0 + _KS1, x1 + (_KS2 + np.uint32(4))
    x0, x1 = _round4(x0, x1, (13, 15, 26, 6))
    x0, x1 = x0 + _KS2, x1 + np.uint32(5)
    return x0 ^ x1


def _uniform_from_bits(bits):
    fbits = (bits >> np.uint32(9)) | np.uint32(0x3F800000)
    floats = lax.bitcast_convert_type(fbits, jnp.float32) - np.float32(1.0)
    return floats + _TINY


def _tc_kernel(logits_ref, val_ref, idx_ref, best_val, best_idx):
    j = pl.program_id(0)
    nblocks = pl.num_programs(0)
    jblk = jnp.where(j == nblocks - 1, TAIL_BLOCK, j)

    lane = lax.broadcasted_iota(jnp.int32, (ROWS, BLOCK_COLS), 1)
    row = lax.broadcasted_iota(jnp.int32, (ROWS, BLOCK_COLS), 0)
    col = jblk * BLOCK_COLS + lane
    m = (row * COLS + col + 42).astype(jnp.uint32)

    u = _uniform_from_bits(_threefry_bits(m))
    g = -jnp.log(-jnp.log(u))

    x = logits_ref[...] + g
    x = jnp.where(col < COLS, x, -jnp.inf)

    bm = jnp.max(x, axis=1, keepdims=True)
    bi = jnp.min(jnp.where(x == bm, col, jnp.int32(2**30)), axis=1,
                 keepdims=True)

    @pl.when(j == 0)
    def _():
        best_val[...] = bm
        best_idx[...] = bi

    @pl.when(j > 0)
    def _():
        upd = bm > best_val[...]
        best_val[...] = jnp.where(upd, bm, best_val[...])
        best_idx[...] = jnp.where(upd, bi, best_idx[...])

    @pl.when(j == nblocks - 1)
    def _():
        val_ref[...] = best_val[...]
        idx_ref[...] = best_idx[...]


def _neg_ln_u(u):
    ub = lax.bitcast_convert_type(u, jnp.int32)
    be = (ub >> np.int32(23)).astype(jnp.float32)
    y = (np.float32(126.5) - be) * _LN2
    for _ in range(4):
        y = y + np.float32(1.0) - u * jnp.exp(y)
    d = np.float32(1.0) - u
    p = jnp.full_like(u, np.float32(1.0 / 13.0))
    for kinv in range(12, 0, -1):
        p = p * d + np.float32(1.0 / kinv)
    return jnp.where(u >= np.float32(0.75), d * p, y)


def _ln_t(t):
    tb = lax.bitcast_convert_type(t, jnp.int32)
    be = (tb >> np.int32(23)).astype(jnp.float32)
    z = (be - np.float32(126.5)) * _LN2
    for _ in range(4):
        z = z - np.float32(1.0) + t * jnp.exp(-z)
    return z


_SC_MESH = plsc.VectorSubcoreMesh(
    core_axis_name="c", subcore_axis_name="s", num_cores=2, num_subcores=16)


@functools.partial(
    pl.kernel,
    out_type=(
        jax.ShapeDtypeStruct((ROWS, 16), jnp.float32),
        jax.ShapeDtypeStruct((ROWS, 16), jnp.float32),
        jax.ShapeDtypeStruct((ROWS, 16), jnp.int32),
        jax.ShapeDtypeStruct((ROWS, 16), jnp.float32),
        jax.ShapeDtypeStruct((ROWS, 16), jnp.float32),
        jax.ShapeDtypeStruct((ROWS, 16), jnp.int32),
    ),
    mesh=_SC_MESH,
    compiler_params=pltpu.CompilerParams(use_tc_tiling_on_sc=True),
    scratch_types=[
        pltpu.VMEM((8, SC_HALF), jnp.float32),
        pltpu.VMEM((8, 16), jnp.float32),
        pltpu.VMEM((8, 16), jnp.float32),
        pltpu.VMEM((8, 16), jnp.int32),
    ],
)
def _sc_scan(logits_hbm, u0, l0, c0, u1, l1, c1, slab, bu, bl, bc):
    wid = lax.axis_index("s") * 2 + lax.axis_index("c")
    grp = wid // 2
    half = wid % 2
    row0 = grp * 8
    cbase = SC_LO + half * SC_HALF
    pltpu.sync_copy(
        logits_hbm.at[pl.ds(row0, 8), pl.ds(half * SC_HALF, SC_HALF)], slab)

    lanes = lax.iota(jnp.int32, 16)
    lanes_u = lanes.astype(jnp.uint32)

    for rr in range(8):
        nbase = (row0 + rr) * COLS + cbase + 42

        def step(k, carry, rr=rr, nbase=nbase, cbase=cbase):
            xb, colb, ub, lb = carry
            base = k * 32
            logit1 = slab[rr, pl.ds(base, 16)]
            logit2 = slab[rr, pl.ds(base + 16, 16)]
            m1 = (nbase + base).astype(jnp.uint32) + lanes_u
            m2 = m1 + np.uint32(16)
            v1 = _uniform_from_bits(_threefry_bits(m1))
            v2 = _uniform_from_bits(_threefry_bits(m2))
            xc1 = logit1 - _ln_t(_neg_ln_u(v1))
            xc2 = logit2 - _ln_t(_neg_ln_u(v2))
            colv1 = (cbase + base) + lanes

            win1 = xc1 > xb
            xb = jnp.where(win1, xc1, xb)
            colb = jnp.where(win1, colv1, colb)
            ub = jnp.where(win1, v1, ub)
            lb = jnp.where(win1, logit1, lb)
            win2 = xc2 > xb
            return (jnp.where(win2, xc2, xb),
                    jnp.where(win2, colv1 + np.int32(16), colb),
                    jnp.where(win2, v2, ub),
                    jnp.where(win2, logit2, lb))

        init = (jnp.full((16,), -3.4e38, jnp.float32),
                jnp.zeros((16,), jnp.int32),
                jnp.full((16,), _TINY, jnp.float32),
                jnp.zeros((16,), jnp.float32))
        xb, colb, ub, lb = lax.fori_loop(0, SC_HALF // 32, step, init)
        bu[rr, :] = ub
        bl[rr, :] = lb
        bc[rr, :] = colb

    @pl.when(half == 0)
    def _():
        pltpu.sync_copy(bu, u0.at[pl.ds(row0, 8), :])
        pltpu.sync_copy(bl, l0.at[pl.ds(row0, 8), :])
        pltpu.sync_copy(bc, c0.at[pl.ds(row0, 8), :])

    @pl.when(half == 1)
    def _():
        pltpu.sync_copy(bu, u1.at[pl.ds(row0, 8), :])
        pltpu.sync_copy(bl, l1.at[pl.ds(row0, 8), :])
        pltpu.sync_copy(bc, c1.at[pl.ds(row0, 8), :])


def _merge_kernel(tcv_ref, tci_ref, u0_ref, l0_ref, c0_ref,
                  u1_ref, l1_ref, c1_ref, out_ref):
    big = jnp.int32(2**30)
    x0 = l0_ref[...] - jnp.log(-jnp.log(u0_ref[...]))
    x1 = l1_ref[...] - jnp.log(-jnp.log(u1_ref[...]))
    bm0 = jnp.max(x0, axis=1, keepdims=True)
    bm1 = jnp.max(x1, axis=1, keepdims=True)
    bm = jnp.maximum(bm0, bm1)
    bi0 = jnp.min(jnp.where(x0 == bm, c0_ref[...], big), axis=1, keepdims=True)
    bi1 = jnp.min(jnp.where(x1 == bm, c1_ref[...], big), axis=1, keepdims=True)
    bi = jnp.minimum(bi0, bi1)

    tv = tcv_ref[...]
    ti = tci_ref[...]
    take_sc = (bm > tv) | ((bm == tv) & (bi < ti))
    out_ref[...] = jnp.where(take_sc, bi, ti)


@jax.jit
def kernel(logits):
    sc_in = lax.slice(logits, (0, SC_LO), (ROWS, SC_HI))
    sc_res = _sc_scan(sc_in)
    tcv, tci = pl.pallas_call(
        _tc_kernel,
        grid=(TC_BLOCKS,),
        in_specs=[pl.BlockSpec(
            (ROWS, BLOCK_COLS),
            lambda j: (0, jnp.where(j == TC_BLOCKS - 1, TAIL_BLOCK, j)))],
        out_specs=(pl.BlockSpec((ROWS, 1), lambda j: (0, 0)),
                   pl.BlockSpec((ROWS, 1), lambda j: (0, 0))),
        out_shape=(jax.ShapeDtypeStruct((ROWS, 1), jnp.float32),
                   jax.ShapeDtypeStruct((ROWS, 1), jnp.int32)),
        scratch_shapes=[
            pltpu.VMEM((ROWS, 1), jnp.float32),
            pltpu.VMEM((ROWS, 1), jnp.int32),
        ],
    )(logits)
    out = pl.pallas_call(
        _merge_kernel,
        out_shape=jax.ShapeDtypeStruct((ROWS, 1), jnp.int32),
    )(tcv, tci, *sc_res)
    return out.reshape(ROWS)
